# Initial kernel scaffold; baseline (speedup 1.0000x reference)
import jax
import jax.numpy as jnp
from jax.experimental import pallas as pl


def kernel(x, x_e, edge_index, node_graph_id, W_atom, W_bond, eps, gin_W1, gin_b1, gin_g1, gin_be1, gin_W2, gin_b2, gin_g2, gin_be2, v_W1, v_b1, v_g1, v_be1, v_W2, v_b2, v_g2, v_be2, v_emb_w, Wp, bp):
    raise NotImplementedError("write your pallas kernel here")


# speedup vs baseline: 2.6648x; 2.6648x over previous
import functools

import jax
import jax.numpy as jnp
import numpy as np
from jax import lax
from jax.experimental import pallas as pl
from jax.experimental.pallas import tpu as pltpu
from jax.experimental.pallas import tpu_sc as plsc

N = 10000
E = 160000
B = 64
H = 256
L = 5
NOUT = 128
NAF = 9
AV = 100
NBF = 3
BV = 5
H2 = 2 * H
HH = H // 2
NB = 10
BN = N // NB
NT = 16
EPT = E // NT
CK = 80
NCH = EPT // CK
ZR = 128
RPS = 640
NP = NT * RPS
NTAB = 125

_F32 = jnp.float32
_HI = lax.Precision.HIGHEST
_RN = float(np.float32(1.0 / N))


def _groupsum(val, acc, scale):
    a = acc
    for k in range(val.shape[0] // 8):
        a = a + val[8 * k:8 * k + 8] * scale
    return a


def _foldmu(sg):
    f = sg[0:4] + sg[4:8]
    f = f[0:2] + f[2:4]
    return f[0:1] + f[1:2]


def _dot(a, b):
    return jnp.dot(a, b, precision=_HI, preferred_element_type=_F32)


def _dotd(a, b):
    return jnp.dot(a, b, preferred_element_type=_F32)


def _atom_body(x_ref, w_ref, out_ref):
    codes = x_ref[...]
    io = lax.broadcasted_iota(jnp.int32, (BN, NAF * AV), 1)
    oh = jnp.zeros((BN, NAF * AV), _F32)
    for f in range(NAF):
        oh = oh + (codes[:, f:f + 1] + f * AV == io).astype(_F32)
    hn = _dot(oh, w_ref[...])
    out_ref[0] = hn[:, :HH]
    out_ref[1] = hn[:, HH:]


def _atom_embed(x, w_flat):
    return pl.pallas_call(
        _atom_body,
        grid=(NB,),
        in_specs=[
            pl.BlockSpec((BN, NAF), lambda i: (i, 0)),
            pl.BlockSpec((NAF * AV, H), lambda i: (0, 0)),
        ],
        out_specs=pl.BlockSpec((2, BN, HH), lambda i: (0, i, 0)),
        out_shape=jax.ShapeDtypeStruct((2, N, HH), _F32),
    )(x, w_flat)


def _codes_body(xe_ref, out_ref):
    out_ref[...] = 25 * xe_ref[0] + 5 * xe_ref[1] + xe_ref[2]


def _edge_codes(xe3):
    return pl.pallas_call(
        _codes_body,
        grid=(1,),
        in_specs=[pl.BlockSpec((NBF, 1250, 128), lambda i: (0, 0, 0))],
        out_specs=pl.BlockSpec((1250, 128), lambda i: (0, 0)),
        out_shape=jax.ShapeDtypeStruct((1250, 128), jnp.int32),
    )(xe3)


def _pre_body(hn_ref, gid_ref, ve_ref, out_ref):
    g = gid_ref[...]
    oh = (g == lax.broadcasted_iota(jnp.int32, (BN, B), 1)).astype(_F32)
    ve = _dot(oh, ve_ref[...])
    out_ref[0] = hn_ref[0] + ve[:, :HH]
    out_ref[1] = hn_ref[1] + ve[:, HH:]


def _pre(hns, gid2d, v_emb):
    return pl.pallas_call(
        _pre_body,
        grid=(NB,),
        in_specs=[
            pl.BlockSpec((2, BN, HH), lambda i: (0, i, 0)),
            pl.BlockSpec((BN, 1), lambda i: (i, 0)),
            pl.BlockSpec((B, H), lambda i: (0, 0)),
        ],
        out_specs=pl.BlockSpec((2, BN, HH), lambda i: (0, i, 0)),
        out_shape=jax.ShapeDtypeStruct((2, N, HH), _F32),
    )(hns, gid2d, v_emb)


def _ka_body(hn_ref, agg_ref, w_ref, b_ref, eps_ref, h1_ref, st_ref):
    ep = 1.0 + eps_ref[0, 0]
    h = jnp.concatenate([ep * hn_ref[0] + agg_ref[0],
                         ep * hn_ref[1] + agg_ref[1]], axis=1)
    h1 = _dotd(h, w_ref[...]) + b_ref[...]
    h1_ref[...] = h1

    @pl.when(pl.program_id(0) == 0)
    def _():
        st_ref[...] = jnp.zeros_like(st_ref)
        st_ref[9:10, :] = jnp.mean(h1, axis=0, keepdims=True)

    sh = st_ref[9:10, :]
    d = h1 - sh
    st_ref[8:9, :] += jnp.sum(d * d, axis=0, keepdims=True)
    st_ref[0:8, :] = _groupsum(h1, st_ref[0:8, :], _RN)


def _ka(hns, aggs, w1, b1, epsv):
    return pl.pallas_call(
        _ka_body,
        grid=(NB,),
        in_specs=[
            pl.BlockSpec((2, BN, HH), lambda i: (0, i, 0)),
            pl.BlockSpec((2, BN, HH), lambda i: (0, i, 0)),
            pl.BlockSpec((H, H2), lambda i: (0, 0)),
            pl.BlockSpec((1, H2), lambda i: (0, 0)),
            pl.BlockSpec(memory_space=pltpu.SMEM),
        ],
        out_specs=[
            pl.BlockSpec((BN, H2), lambda i: (i, 0)),
            pl.BlockSpec((10, H2), lambda i: (0, 0)),
        ],
        out_shape=[
            jax.ShapeDtypeStruct((N, H2), _F32),
            jax.ShapeDtypeStruct((10, H2), _F32),
        ],
    )(hns, aggs, w1, b1, epsv)


def _kb_body(h1_ref, st_ref, g_ref, be_ref, w_ref, b_ref, h2_ref, st2_ref):
    s = st_ref[...]
    mu = _foldmu(s[0:8])
    dmu = mu - s[9:10, :]
    var = s[8:9, :] / N - dmu * dmu
    y = jnp.maximum((h1_ref[...] - mu) / jnp.sqrt(var + 1e-5) * g_ref[...]
                    + be_ref[...], 0.0)
    h2 = _dotd(y, w_ref[...]) + b_ref[...]
    h2_ref[...] = h2

    @pl.when(pl.program_id(0) == 0)
    def _():
        st2_ref[...] = jnp.zeros_like(st2_ref)
        st2_ref[9:10, :] = jnp.mean(h2, axis=0, keepdims=True)

    sh = st2_ref[9:10, :]
    d = h2 - sh
    st2_ref[8:9, :] += jnp.sum(d * d, axis=0, keepdims=True)
    st2_ref[0:8, :] = _groupsum(h2, st2_ref[0:8, :], _RN)


def _kb(h1, st, g1, be1, w2, b2):
    return pl.pallas_call(
        _kb_body,
        grid=(NB,),
        in_specs=[
            pl.BlockSpec((BN, H2), lambda i: (i, 0)),
            pl.BlockSpec((10, H2), lambda i: (0, 0)),
            pl.BlockSpec((1, H2), lambda i: (0, 0)),
            pl.BlockSpec((1, H2), lambda i: (0, 0)),
            pl.BlockSpec((H2, H), lambda i: (0, 0)),
            pl.BlockSpec((1, H), lambda i: (0, 0)),
        ],
        out_specs=[
            pl.BlockSpec((BN, H), lambda i: (i, 0)),
            pl.BlockSpec((10, H), lambda i: (0, 0)),
        ],
        out_shape=[
            jax.ShapeDtypeStruct((N, H), _F32),
            jax.ShapeDtypeStruct((10, H), _F32),
        ],
    )(h1, st, g1, be1, w2, b2)


def _kc_body(h2_ref, st_ref, g_ref, be_ref, gid_ref, hn_ref, pool_ref, cnt_ref):
    s = st_ref[...]
    mu = _foldmu(s[0:8])
    dmu = mu - s[9:10, :]
    var = s[8:9, :] / N - dmu * dmu
    y = jnp.maximum((h2_ref[...] - mu) / jnp.sqrt(var + 1e-5) * g_ref[...]
                    + be_ref[...], 0.0)
    hn_ref[0] = y[:, :HH]
    hn_ref[1] = y[:, HH:]
    oh = (gid_ref[...] == lax.broadcasted_iota(jnp.int32, (BN, B), 1)).astype(_F32)

    @pl.when(pl.program_id(0) == 0)
    def _():
        pool_ref[...] = jnp.zeros_like(pool_ref)
        cnt_ref[...] = jnp.zeros_like(cnt_ref)

    pool_ref[...] += lax.dot_general(oh, y, (((0,), (0,)), ((), ())),
                                     precision=_HI, preferred_element_type=_F32)
    cnt_ref[...] += jnp.sum(oh, axis=0, keepdims=True)


def _kc(h2, st2, g2, be2, gid2d):
    return pl.pallas_call(
        _kc_body,
        grid=(NB,),
        in_specs=[
            pl.BlockSpec((BN, H), lambda i: (i, 0)),
            pl.BlockSpec((10, H), lambda i: (0, 0)),
            pl.BlockSpec((1, H), lambda i: (0, 0)),
            pl.BlockSpec((1, H), lambda i: (0, 0)),
            pl.BlockSpec((BN, 1), lambda i: (i, 0)),
        ],
        out_specs=[
            pl.BlockSpec((2, BN, HH), lambda i: (0, i, 0)),
            pl.BlockSpec((B, H), lambda i: (0, 0)),
            pl.BlockSpec((1, B), lambda i: (0, 0)),
        ],
        out_shape=[
            jax.ShapeDtypeStruct((2, N, HH), _F32),
            jax.ShapeDtypeStruct((B, H), _F32),
            jax.ShapeDtypeStruct((1, B), _F32),
        ],
    )(h2, st2, g2, be2, gid2d)


def _kv_body(p_ref, ve_ref, w1_ref, b1_ref, g1_ref, be1_ref,
             w2_ref, b2_ref, g2_ref, be2_ref, out_ref):
    v = p_ref[...] + ve_ref[...]
    a = _dotd(v, w1_ref[...]) + b1_ref[...]
    mu = jnp.mean(a, axis=0, keepdims=True)
    d = a - mu
    var = jnp.mean(d * d, axis=0, keepdims=True)
    a = jnp.maximum(d / jnp.sqrt(var + 1e-5) * g1_ref[...] + be1_ref[...], 0.0)
    a = _dotd(a, w2_ref[...]) + b2_ref[...]
    mu = jnp.mean(a, axis=0, keepdims=True)
    d = a - mu
    var = jnp.mean(d * d, axis=0, keepdims=True)
    a = d / jnp.sqrt(var + 1e-5) * g2_ref[...] + be2_ref[...]
    out_ref[...] = jnp.maximum(a, 0.0)


def _kv(pooled, v_emb, w1, b1, g1, be1, w2, b2, g2, be2):
    specs = [
        pl.BlockSpec((B, H), lambda: (0, 0)),
        pl.BlockSpec((B, H), lambda: (0, 0)),
        pl.BlockSpec((H, H2), lambda: (0, 0)),
        pl.BlockSpec((1, H2), lambda: (0, 0)),
        pl.BlockSpec((1, H2), lambda: (0, 0)),
        pl.BlockSpec((1, H2), lambda: (0, 0)),
        pl.BlockSpec((H2, H), lambda: (0, 0)),
        pl.BlockSpec((1, H), lambda: (0, 0)),
        pl.BlockSpec((1, H), lambda: (0, 0)),
        pl.BlockSpec((1, H), lambda: (0, 0)),
    ]
    return pl.pallas_call(
        _kv_body,
        in_specs=specs,
        out_specs=pl.BlockSpec((B, H), lambda: (0, 0)),
        out_shape=jax.ShapeDtypeStruct((B, H), _F32),
    )(pooled, v_emb, w1, b1, g1, be1, w2, b2, g2, be2)


def _kf_body(p_ref, c_ref, w_ref, b_ref, out_ref):
    r = 1.0 / jnp.maximum(c_ref[...], 1.0)
    eye = (lax.broadcasted_iota(jnp.int32, (B, B), 0)
           == lax.broadcasted_iota(jnp.int32, (B, B), 1)).astype(_F32)
    pm = _dot(eye * r, p_ref[...])
    out_ref[...] = _dotd(pm, w_ref[...]) + b_ref[...]


def _kf(pooled, counts, wp, bp):
    return pl.pallas_call(
        _kf_body,
        in_specs=[
            pl.BlockSpec((B, H), lambda: (0, 0)),
            pl.BlockSpec((1, B), lambda: (0, 0)),
            pl.BlockSpec((H, NOUT), lambda: (0, 0)),
            pl.BlockSpec((1, NOUT), lambda: (0, 0)),
        ],
        out_specs=pl.BlockSpec((B, NOUT), lambda: (0, 0)),
        out_shape=jax.ShapeDtypeStruct((B, NOUT), _F32),
    )(pooled, counts, wp, bp)


def _edge_body(hn_hbm, t_hbm, src_hbm, dst_hbm, code_hbm, out_hbm,
               acc, srcv, dstv, codev, hrow, trow, zbuf, sem1, sem2):
    c = lax.axis_index("c")
    s = lax.axis_index("s")
    coff = c * N
    toff = c * NTAB

    def _zb(i, carry):
        r = i // 8
        k = (i % 8) * 16
        zbuf[r, pl.ds(k, 16)] = jnp.zeros((16,), _F32)
        return carry

    lax.fori_loop(0, ZR * 8, _zb, 0)

    def _zc(i, carry):
        pltpu.sync_copy(zbuf, acc.at[pl.ds(s * RPS + i * ZR, ZR)])
        return carry

    lax.fori_loop(0, RPS // ZR, _zc, 0)
    plsc.subcore_barrier()

    def _chunk(j, carry):
        base = s * EPT + j * CK
        pltpu.sync_copy(src_hbm.at[pl.ds(base, CK)], srcv)
        pltpu.sync_copy(dst_hbm.at[pl.ds(base, CK)], dstv)
        pltpu.sync_copy(code_hbm.at[pl.ds(base, CK)], codev)

        def _adj(g, cr):
            srcv[pl.ds(g * 16, 16)] = srcv[pl.ds(g * 16, 16)] + coff
            codev[pl.ds(g * 16, 16)] = codev[pl.ds(g * 16, 16)] + toff
            return cr

        lax.fori_loop(0, CK // 16, _adj, 0)
        cp1 = pltpu.async_copy(hn_hbm.at[srcv], hrow, sem1)
        cp2 = pltpu.async_copy(t_hbm.at[codev], trow, sem2)
        cp1.wait()
        cp2.wait()

        def _cmp(v, cr):
            e = v // 8
            k = (v % 8) * 16
            hrow[e, pl.ds(k, 16)] = jnp.maximum(
                hrow[e, pl.ds(k, 16)] + trow[e, pl.ds(k, 16)], 0.0)
            return cr

        lax.fori_loop(0, CK * 8, _cmp, 0)
        pltpu.sync_copy(hrow, acc.at[dstv], add=True)
        return carry

    lax.fori_loop(0, NCH, _chunk, 0)
    plsc.subcore_barrier()
    pltpu.sync_copy(acc.at[pl.ds(s * RPS, RPS)],
                    out_hbm.at[pl.ds(c * NP + s * RPS, RPS)])


@functools.cache
def _edge_kernel():
    mesh = plsc.VectorSubcoreMesh(core_axis_name="c", subcore_axis_name="s")
    return pl.kernel(
        _edge_body,
        out_type=jax.ShapeDtypeStruct((2 * NP, HH), _F32),
        mesh=mesh,
        scratch_types=[
            pltpu.VMEM_SHARED((NP, HH), _F32),
            pltpu.VMEM((CK,), jnp.int32),
            pltpu.VMEM((CK,), jnp.int32),
            pltpu.VMEM((CK,), jnp.int32),
            pltpu.VMEM((CK, HH), _F32),
            pltpu.VMEM((CK, HH), _F32),
            pltpu.VMEM((ZR, HH), _F32),
            pltpu.SemaphoreType.DMA,
            pltpu.SemaphoreType.DMA,
        ],
    )


def _edge_call(hnflat, tab, src, dst, codes):
    return _edge_kernel()(hnflat, tab, src, dst, codes)


def kernel(x, x_e, edge_index, node_graph_id, W_atom, W_bond, eps,
           gin_W1, gin_b1, gin_g1, gin_be1, gin_W2, gin_b2, gin_g2, gin_be2,
           v_W1, v_b1, v_g1, v_be1, v_W2, v_b2, v_g2, v_be2,
           v_emb_w, Wp, bp):
    x = x.astype(jnp.int32)
    x_e = x_e.astype(jnp.int32)
    src = edge_index[0].astype(jnp.int32)
    dst = edge_index[1].astype(jnp.int32)
    gid2d = node_graph_id.astype(jnp.int32).reshape(N, 1)

    w_atom_flat = W_atom.reshape(NAF * AV, H)
    cc = jnp.arange(NTAB)
    tab = (W_bond[:, 0, cc // 25] + W_bond[:, 1, (cc // 5) % 5]
           + W_bond[:, 2, cc % 5])
    tab_split = tab.reshape(L, NTAB, 2, HH).transpose(0, 2, 1, 3)
    tab_split = tab_split.reshape(L, 2 * NTAB, HH)

    codes = _edge_codes(x_e.T.reshape(NBF, 1250, 128)).reshape(E)

    hns = _atom_embed(x, w_atom_flat)
    v_emb = jnp.broadcast_to(v_emb_w, (B, H))

    pooled = None
    counts = None
    for i in range(L):
        hn2s = _pre(hns, gid2d, v_emb)
        aggf = _edge_call(hn2s.reshape(2 * N, HH), tab_split[i], src, dst, codes)
        aggs = aggf.reshape(2, NP, HH)
        h1, st1 = _ka(hn2s, aggs, gin_W1[i], gin_b1[i].reshape(1, H2),
                      eps[i].reshape(1, 1))
        h2, st2 = _kb(h1, st1, gin_g1[i].reshape(1, H2),
                      gin_be1[i].reshape(1, H2), gin_W2[i],
                      gin_b2[i].reshape(1, H))
        hns, pooled, counts = _kc(h2, st2, gin_g2[i].reshape(1, H),
                                  gin_be2[i].reshape(1, H), gid2d)
        if i < L - 1:
            v_emb = _kv(pooled, v_emb, v_W1[i], v_b1[i].reshape(1, H2),
                        v_g1[i].reshape(1, H2), v_be1[i].reshape(1, H2),
                        v_W2[i], v_b2[i].reshape(1, H),
                        v_g2[i].reshape(1, H), v_be2[i].reshape(1, H))

    return _kf(pooled, counts, Wp, bp.reshape(1, NOUT))


# speedup vs baseline: 5.9343x; 2.2269x over previous
import functools

import jax
import jax.numpy as jnp
import numpy as np
from jax import lax
from jax.experimental import pallas as pl
from jax.experimental.pallas import tpu as pltpu
from jax.experimental.pallas import tpu_sc as plsc

N = 10000
E = 160000
B = 64
H = 256
L = 5
NOUT = 128
NAF = 9
AV = 100
NBF = 3
BV = 5
H2 = 2 * H
HH = H // 2
NB = 10
BN = N // NB
NT = 16
EPT = E // NT
CK = 80
NCH = EPT // CK
ZR = 32
RPS = 640
NP = NT * RPS
NTAB = 125

_F32 = jnp.float32
_HI = lax.Precision.HIGHEST
_RN = float(np.float32(1.0 / N))


def _groupsum(val, acc, scale):
    a = acc
    for k in range(val.shape[0] // 8):
        a = a + val[8 * k:8 * k + 8] * scale
    return a


def _foldmu(sg):
    f = sg[0:4] + sg[4:8]
    f = f[0:2] + f[2:4]
    return f[0:1] + f[1:2]


def _dot(a, b):
    return jnp.dot(a, b, precision=_HI, preferred_element_type=_F32)


def _dotd(a, b):
    return jnp.dot(a, b, preferred_element_type=_F32)


def _atom_body(x_ref, w_ref, out_ref):
    codes = x_ref[...]
    io = lax.broadcasted_iota(jnp.int32, (BN, NAF * AV), 1)
    oh = jnp.zeros((BN, NAF * AV), _F32)
    for f in range(NAF):
        oh = oh + (codes[:, f:f + 1] + f * AV == io).astype(_F32)
    hn = _dot(oh, w_ref[...])
    out_ref[0] = hn[:, :HH]
    out_ref[1] = hn[:, HH:]


def _atom_embed(x, w_flat):
    return pl.pallas_call(
        _atom_body,
        grid=(NB,),
        in_specs=[
            pl.BlockSpec((BN, NAF), lambda i: (i, 0)),
            pl.BlockSpec((NAF * AV, H), lambda i: (0, 0)),
        ],
        out_specs=pl.BlockSpec((2, BN, HH), lambda i: (0, i, 0)),
        out_shape=jax.ShapeDtypeStruct((2, N, HH), _F32),
    )(x, w_flat)


def _codes_body(xe_ref, out_ref):
    out_ref[...] = 25 * xe_ref[0] + 5 * xe_ref[1] + xe_ref[2]


def _edge_codes(xe3):
    return pl.pallas_call(
        _codes_body,
        grid=(1,),
        in_specs=[pl.BlockSpec((NBF, 1250, 128), lambda i: (0, 0, 0))],
        out_specs=pl.BlockSpec((1250, 128), lambda i: (0, 0)),
        out_shape=jax.ShapeDtypeStruct((1250, 128), jnp.int32),
    )(xe3)


def _pre_body(hn_ref, gid_ref, ve_ref, out_ref):
    g = gid_ref[...]
    oh = (g == lax.broadcasted_iota(jnp.int32, (BN, B), 1)).astype(_F32)
    ve = _dot(oh, ve_ref[...])
    out_ref[0] = hn_ref[0] + ve[:, :HH]
    out_ref[1] = hn_ref[1] + ve[:, HH:]


def _pre(hns, gid2d, v_emb):
    return pl.pallas_call(
        _pre_body,
        grid=(NB,),
        in_specs=[
            pl.BlockSpec((2, BN, HH), lambda i: (0, i, 0)),
            pl.BlockSpec((BN, 1), lambda i: (i, 0)),
            pl.BlockSpec((B, H), lambda i: (0, 0)),
        ],
        out_specs=pl.BlockSpec((2, BN, HH), lambda i: (0, i, 0)),
        out_shape=jax.ShapeDtypeStruct((2, N, HH), _F32),
    )(hns, gid2d, v_emb)


def _ka_body(hn_ref, agg_ref, w_ref, b_ref, eps_ref, h1_ref, st_ref):
    ep = 1.0 + eps_ref[0, 0]
    h = jnp.concatenate([ep * hn_ref[0] + agg_ref[0],
                         ep * hn_ref[1] + agg_ref[1]], axis=1)
    h1 = _dotd(h, w_ref[...]) + b_ref[...]
    h1_ref[...] = h1

    @pl.when(pl.program_id(0) == 0)
    def _():
        st_ref[...] = jnp.zeros_like(st_ref)
        st_ref[9:10, :] = jnp.mean(h1, axis=0, keepdims=True)

    sh = st_ref[9:10, :]
    d = h1 - sh
    st_ref[8:9, :] += jnp.sum(d * d, axis=0, keepdims=True)
    st_ref[0:8, :] = _groupsum(h1, st_ref[0:8, :], _RN)


def _ka(hns, aggs, w1, b1, epsv):
    return pl.pallas_call(
        _ka_body,
        grid=(NB,),
        in_specs=[
            pl.BlockSpec((2, BN, HH), lambda i: (0, i, 0)),
            pl.BlockSpec((2, BN, HH), lambda i: (0, i, 0)),
            pl.BlockSpec((H, H2), lambda i: (0, 0)),
            pl.BlockSpec((1, H2), lambda i: (0, 0)),
            pl.BlockSpec(memory_space=pltpu.SMEM),
        ],
        out_specs=[
            pl.BlockSpec((BN, H2), lambda i: (i, 0)),
            pl.BlockSpec((10, H2), lambda i: (0, 0)),
        ],
        out_shape=[
            jax.ShapeDtypeStruct((N, H2), _F32),
            jax.ShapeDtypeStruct((10, H2), _F32),
        ],
    )(hns, aggs, w1, b1, epsv)


def _kb_body(h1_ref, st_ref, g_ref, be_ref, w_ref, b_ref, h2_ref, st2_ref):
    s = st_ref[...]
    mu = _foldmu(s[0:8])
    dmu = mu - s[9:10, :]
    var = s[8:9, :] / N - dmu * dmu
    y = jnp.maximum((h1_ref[...] - mu) / jnp.sqrt(var + 1e-5) * g_ref[...]
                    + be_ref[...], 0.0)
    h2 = _dotd(y, w_ref[...]) + b_ref[...]
    h2_ref[...] = h2

    @pl.when(pl.program_id(0) == 0)
    def _():
        st2_ref[...] = jnp.zeros_like(st2_ref)
        st2_ref[9:10, :] = jnp.mean(h2, axis=0, keepdims=True)

    sh = st2_ref[9:10, :]
    d = h2 - sh
    st2_ref[8:9, :] += jnp.sum(d * d, axis=0, keepdims=True)
    st2_ref[0:8, :] = _groupsum(h2, st2_ref[0:8, :], _RN)


def _kb(h1, st, g1, be1, w2, b2):
    return pl.pallas_call(
        _kb_body,
        grid=(NB,),
        in_specs=[
            pl.BlockSpec((BN, H2), lambda i: (i, 0)),
            pl.BlockSpec((10, H2), lambda i: (0, 0)),
            pl.BlockSpec((1, H2), lambda i: (0, 0)),
            pl.BlockSpec((1, H2), lambda i: (0, 0)),
            pl.BlockSpec((H2, H), lambda i: (0, 0)),
            pl.BlockSpec((1, H), lambda i: (0, 0)),
        ],
        out_specs=[
            pl.BlockSpec((BN, H), lambda i: (i, 0)),
            pl.BlockSpec((10, H), lambda i: (0, 0)),
        ],
        out_shape=[
            jax.ShapeDtypeStruct((N, H), _F32),
            jax.ShapeDtypeStruct((10, H), _F32),
        ],
    )(h1, st, g1, be1, w2, b2)


def _kc_body(h2_ref, st_ref, g_ref, be_ref, gid_ref, hn_ref, pool_ref, cnt_ref):
    s = st_ref[...]
    mu = _foldmu(s[0:8])
    dmu = mu - s[9:10, :]
    var = s[8:9, :] / N - dmu * dmu
    y = jnp.maximum((h2_ref[...] - mu) / jnp.sqrt(var + 1e-5) * g_ref[...]
                    + be_ref[...], 0.0)
    hn_ref[0] = y[:, :HH]
    hn_ref[1] = y[:, HH:]
    oh = (gid_ref[...] == lax.broadcasted_iota(jnp.int32, (BN, B), 1)).astype(_F32)

    @pl.when(pl.program_id(0) == 0)
    def _():
        pool_ref[...] = jnp.zeros_like(pool_ref)
        cnt_ref[...] = jnp.zeros_like(cnt_ref)

    pool_ref[...] += lax.dot_general(oh, y, (((0,), (0,)), ((), ())),
                                     precision=_HI, preferred_element_type=_F32)
    cnt_ref[...] += jnp.sum(oh, axis=0, keepdims=True)


def _kc(h2, st2, g2, be2, gid2d):
    return pl.pallas_call(
        _kc_body,
        grid=(NB,),
        in_specs=[
            pl.BlockSpec((BN, H), lambda i: (i, 0)),
            pl.BlockSpec((10, H), lambda i: (0, 0)),
            pl.BlockSpec((1, H), lambda i: (0, 0)),
            pl.BlockSpec((1, H), lambda i: (0, 0)),
            pl.BlockSpec((BN, 1), lambda i: (i, 0)),
        ],
        out_specs=[
            pl.BlockSpec((2, BN, HH), lambda i: (0, i, 0)),
            pl.BlockSpec((B, H), lambda i: (0, 0)),
            pl.BlockSpec((1, B), lambda i: (0, 0)),
        ],
        out_shape=[
            jax.ShapeDtypeStruct((2, N, HH), _F32),
            jax.ShapeDtypeStruct((B, H), _F32),
            jax.ShapeDtypeStruct((1, B), _F32),
        ],
    )(h2, st2, g2, be2, gid2d)


def _kv_body(p_ref, ve_ref, w1_ref, b1_ref, g1_ref, be1_ref,
             w2_ref, b2_ref, g2_ref, be2_ref, out_ref):
    v = p_ref[...] + ve_ref[...]
    a = _dotd(v, w1_ref[...]) + b1_ref[...]
    mu = jnp.mean(a, axis=0, keepdims=True)
    d = a - mu
    var = jnp.mean(d * d, axis=0, keepdims=True)
    a = jnp.maximum(d / jnp.sqrt(var + 1e-5) * g1_ref[...] + be1_ref[...], 0.0)
    a = _dotd(a, w2_ref[...]) + b2_ref[...]
    mu = jnp.mean(a, axis=0, keepdims=True)
    d = a - mu
    var = jnp.mean(d * d, axis=0, keepdims=True)
    a = d / jnp.sqrt(var + 1e-5) * g2_ref[...] + be2_ref[...]
    out_ref[...] = jnp.maximum(a, 0.0)


def _kv(pooled, v_emb, w1, b1, g1, be1, w2, b2, g2, be2):
    specs = [
        pl.BlockSpec((B, H), lambda: (0, 0)),
        pl.BlockSpec((B, H), lambda: (0, 0)),
        pl.BlockSpec((H, H2), lambda: (0, 0)),
        pl.BlockSpec((1, H2), lambda: (0, 0)),
        pl.BlockSpec((1, H2), lambda: (0, 0)),
        pl.BlockSpec((1, H2), lambda: (0, 0)),
        pl.BlockSpec((H2, H), lambda: (0, 0)),
        pl.BlockSpec((1, H), lambda: (0, 0)),
        pl.BlockSpec((1, H), lambda: (0, 0)),
        pl.BlockSpec((1, H), lambda: (0, 0)),
    ]
    return pl.pallas_call(
        _kv_body,
        in_specs=specs,
        out_specs=pl.BlockSpec((B, H), lambda: (0, 0)),
        out_shape=jax.ShapeDtypeStruct((B, H), _F32),
    )(pooled, v_emb, w1, b1, g1, be1, w2, b2, g2, be2)


def _kf_body(p_ref, c_ref, w_ref, b_ref, out_ref):
    r = 1.0 / jnp.maximum(c_ref[...], 1.0)
    eye = (lax.broadcasted_iota(jnp.int32, (B, B), 0)
           == lax.broadcasted_iota(jnp.int32, (B, B), 1)).astype(_F32)
    pm = _dot(eye * r, p_ref[...])
    out_ref[...] = _dotd(pm, w_ref[...]) + b_ref[...]


def _kf(pooled, counts, wp, bp):
    return pl.pallas_call(
        _kf_body,
        in_specs=[
            pl.BlockSpec((B, H), lambda: (0, 0)),
            pl.BlockSpec((1, B), lambda: (0, 0)),
            pl.BlockSpec((H, NOUT), lambda: (0, 0)),
            pl.BlockSpec((1, NOUT), lambda: (0, 0)),
        ],
        out_specs=pl.BlockSpec((B, NOUT), lambda: (0, 0)),
        out_shape=jax.ShapeDtypeStruct((B, NOUT), _F32),
    )(pooled, counts, wp, bp)


def _edge_body(hn_hbm, t_hbm, src_hbm, dst_hbm, code_hbm, out_hbm,
               acc, srcv0, dstv0, codev0, srcv1, dstv1, codev1,
               hrow0, trow0, hrow1, trow1, zbuf,
               sh0, st0, sh1, st1):
    c = lax.axis_index("c")
    s = lax.axis_index("s")
    hnc = hn_hbm.at[c]
    tc_ = t_hbm.at[c]

    def _zb(i, carry):
        r = i // 8
        k = (i % 8) * 16
        zbuf[r, pl.ds(k, 16)] = jnp.zeros((16,), _F32)
        return carry

    lax.fori_loop(0, ZR * 8, _zb, 0)

    def _zc(i, carry):
        pltpu.sync_copy(zbuf, acc.at[pl.ds(s * RPS + i * ZR, ZR)])
        return carry

    lax.fori_loop(0, RPS // ZR, _zc, 0)
    plsc.subcore_barrier()

    bufs = ((hrow0, trow0, sh0, st0, srcv0, dstv0, codev0),
            (hrow1, trow1, sh1, st1, srcv1, dstv1, codev1))

    def _issue(j, b):
        hr, tr, sh, st, sv, dv, cv = bufs[b]
        base = s * EPT + j * CK
        pltpu.sync_copy(src_hbm.at[pl.ds(base, CK)], sv)
        pltpu.sync_copy(code_hbm.at[pl.ds(base, CK)], cv)
        pltpu.sync_copy(dst_hbm.at[pl.ds(base, CK)], dv)
        pltpu.async_copy(hnc.at[sv], hr, sh)
        pltpu.async_copy(tc_.at[cv], tr, st)

    def _drain(j, b):
        hr, tr, sh, st, sv, dv, cv = bufs[b]
        pltpu.make_async_copy(hnc.at[sv], hr, sh).wait()
        pltpu.make_async_copy(tc_.at[cv], tr, st).wait()

        def _ce(e, cr):
            for g in range(8):
                sl = pl.ds(g * 16, 16)
                hr[e, sl] = jnp.maximum(hr[e, sl] + tr[e, sl], 0.0)
            return cr

        lax.fori_loop(0, CK, _ce, 0)
        pltpu.sync_copy(hr, acc.at[dv], add=True)

    _issue(0, 0)

    def _pair(i, carry):
        _issue(2 * i + 1, 1)
        _drain(2 * i, 0)
        _issue(2 * i + 2, 0)
        _drain(2 * i + 1, 1)
        return carry

    lax.fori_loop(0, (NCH - 1) // 2, _pair, 0)
    _drain(NCH - 1, 0)

    plsc.subcore_barrier()
    pltpu.sync_copy(acc.at[pl.ds(s * RPS, RPS)],
                    out_hbm.at[pl.ds(c * NP + s * RPS, RPS)])


@functools.cache
def _edge_kernel():
    mesh = plsc.VectorSubcoreMesh(core_axis_name="c", subcore_axis_name="s")
    return pl.kernel(
        _edge_body,
        out_type=jax.ShapeDtypeStruct((2 * NP, HH), _F32),
        mesh=mesh,
        scratch_types=[
            pltpu.VMEM_SHARED((NP, HH), _F32),
            pltpu.VMEM((CK,), jnp.int32),
            pltpu.VMEM((CK,), jnp.int32),
            pltpu.VMEM((CK,), jnp.int32),
            pltpu.VMEM((CK,), jnp.int32),
            pltpu.VMEM((CK,), jnp.int32),
            pltpu.VMEM((CK,), jnp.int32),
            pltpu.VMEM((CK, HH), _F32),
            pltpu.VMEM((CK, HH), _F32),
            pltpu.VMEM((CK, HH), _F32),
            pltpu.VMEM((CK, HH), _F32),
            pltpu.VMEM((ZR, HH), _F32),
            pltpu.SemaphoreType.DMA,
            pltpu.SemaphoreType.DMA,
            pltpu.SemaphoreType.DMA,
            pltpu.SemaphoreType.DMA,
        ],
    )


def _edge_call(hn3, tab3, src3, dst3, code3):
    return _edge_kernel()(hn3, tab3, src3, dst3, code3)


def kernel(x, x_e, edge_index, node_graph_id, W_atom, W_bond, eps,
           gin_W1, gin_b1, gin_g1, gin_be1, gin_W2, gin_b2, gin_g2, gin_be2,
           v_W1, v_b1, v_g1, v_be1, v_W2, v_b2, v_g2, v_be2,
           v_emb_w, Wp, bp):
    x = x.astype(jnp.int32)
    x_e = x_e.astype(jnp.int32)
    src = edge_index[0].astype(jnp.int32)
    dst = edge_index[1].astype(jnp.int32)
    gid2d = node_graph_id.astype(jnp.int32).reshape(N, 1)

    w_atom_flat = W_atom.reshape(NAF * AV, H)
    cc = jnp.arange(NTAB)
    tab = (W_bond[:, 0, cc // 25] + W_bond[:, 1, (cc // 5) % 5]
           + W_bond[:, 2, cc % 5])
    tab_split = tab.reshape(L, NTAB, 2, HH).transpose(0, 2, 1, 3)

    codes = _edge_codes(x_e.T.reshape(NBF, 1250, 128)).reshape(E)

    hns = _atom_embed(x, w_atom_flat)
    v_emb = jnp.broadcast_to(v_emb_w, (B, H))

    pooled = None
    counts = None
    for i in range(L):
        hn2s = _pre(hns, gid2d, v_emb)
        aggf = _edge_call(hn2s, tab_split[i], src, dst, codes)
        aggs = aggf.reshape(2, NP, HH)
        h1, st1 = _ka(hn2s, aggs, gin_W1[i], gin_b1[i].reshape(1, H2),
                      eps[i].reshape(1, 1))
        h2, st2 = _kb(h1, st1, gin_g1[i].reshape(1, H2),
                      gin_be1[i].reshape(1, H2), gin_W2[i],
                      gin_b2[i].reshape(1, H))
        hns, pooled, counts = _kc(h2, st2, gin_g2[i].reshape(1, H),
                                  gin_be2[i].reshape(1, H), gid2d)
        if i < L - 1:
            v_emb = _kv(pooled, v_emb, v_W1[i], v_b1[i].reshape(1, H2),
                        v_g1[i].reshape(1, H2), v_be1[i].reshape(1, H2),
                        v_W2[i], v_b2[i].reshape(1, H),
                        v_g2[i].reshape(1, H), v_be2[i].reshape(1, H))

    return _kf(pooled, counts, Wp, bp.reshape(1, NOUT))


# speedup vs baseline: 5.9432x; 1.0015x over previous
import functools

import jax
import jax.numpy as jnp
import numpy as np
from jax import lax
from jax.experimental import pallas as pl
from jax.experimental.pallas import tpu as pltpu
from jax.experimental.pallas import tpu_sc as plsc

N = 10000
E = 160000
B = 64
H = 256
L = 5
NOUT = 128
NAF = 9
AV = 100
NBF = 3
BV = 5
H2 = 2 * H
HH = H // 2
NB = 10
BN = N // NB
NT = 16
EPT = E // NT
CK = 80
NCH = EPT // CK
ZR = 32
RPS = 640
NP = NT * RPS
NTAB = 125

_F32 = jnp.float32
_HI = lax.Precision.HIGHEST
_RN = float(np.float32(1.0 / N))


def _groupsum(val, acc, scale):
    a = acc
    for k in range(val.shape[0] // 8):
        a = a + val[8 * k:8 * k + 8] * scale
    return a


def _foldmu(sg):
    f = sg[0:4] + sg[4:8]
    f = f[0:2] + f[2:4]
    return f[0:1] + f[1:2]


def _dot(a, b):
    return jnp.dot(a, b, precision=_HI, preferred_element_type=_F32)


def _dotd(a, b):
    return jnp.dot(a, b, preferred_element_type=_F32)


def _atom_body(x_ref, w_ref, out_ref):
    codes = x_ref[...]
    io = lax.broadcasted_iota(jnp.int32, (BN, NAF * AV), 1)
    oh = jnp.zeros((BN, NAF * AV), _F32)
    for f in range(NAF):
        oh = oh + (codes[:, f:f + 1] + f * AV == io).astype(_F32)
    hn = _dot(oh, w_ref[...])
    out_ref[0] = hn[:, :HH]
    out_ref[1] = hn[:, HH:]


def _atom_embed(x, w_flat):
    return pl.pallas_call(
        _atom_body,
        grid=(NB,),
        in_specs=[
            pl.BlockSpec((BN, NAF), lambda i: (i, 0)),
            pl.BlockSpec((NAF * AV, H), lambda i: (0, 0)),
        ],
        out_specs=pl.BlockSpec((2, BN, HH), lambda i: (0, i, 0)),
        out_shape=jax.ShapeDtypeStruct((2, N, HH), _F32),
    )(x, w_flat)


def _codes_body(xe_ref, out_ref):
    out_ref[...] = 25 * xe_ref[0] + 5 * xe_ref[1] + xe_ref[2]


def _edge_codes(xe3):
    return pl.pallas_call(
        _codes_body,
        grid=(1,),
        in_specs=[pl.BlockSpec((NBF, 1250, 128), lambda i: (0, 0, 0))],
        out_specs=pl.BlockSpec((1250, 128), lambda i: (0, 0)),
        out_shape=jax.ShapeDtypeStruct((1250, 128), jnp.int32),
    )(xe3)


def _pre_body(hn_ref, gid_ref, ve_ref, out_ref):
    g = gid_ref[...]
    oh = (g == lax.broadcasted_iota(jnp.int32, (BN, B), 1)).astype(_F32)
    ve = _dot(oh, ve_ref[...])
    out_ref[0] = hn_ref[0] + ve[:, :HH]
    out_ref[1] = hn_ref[1] + ve[:, HH:]


def _pre(hns, gid2d, v_emb):
    return pl.pallas_call(
        _pre_body,
        grid=(NB,),
        in_specs=[
            pl.BlockSpec((2, BN, HH), lambda i: (0, i, 0)),
            pl.BlockSpec((BN, 1), lambda i: (i, 0)),
            pl.BlockSpec((B, H), lambda i: (0, 0)),
        ],
        out_specs=pl.BlockSpec((2, BN, HH), lambda i: (0, i, 0)),
        out_shape=jax.ShapeDtypeStruct((2, N, HH), _F32),
    )(hns, gid2d, v_emb)


def _ka_body(hn_ref, agg_ref, w_ref, b_ref, eps_ref, h1_ref, st_ref):
    ep = 1.0 + eps_ref[0, 0]
    h = jnp.concatenate([ep * hn_ref[0] + agg_ref[0],
                         ep * hn_ref[1] + agg_ref[1]], axis=1)
    h1 = _dotd(h, w_ref[...]) + b_ref[...]
    h1_ref[...] = h1

    @pl.when(pl.program_id(0) == 0)
    def _():
        st_ref[...] = jnp.zeros_like(st_ref)
        st_ref[9:10, :] = jnp.mean(h1, axis=0, keepdims=True)

    sh = st_ref[9:10, :]
    d = h1 - sh
    st_ref[8:9, :] += jnp.sum(d * d, axis=0, keepdims=True)
    st_ref[0:8, :] = _groupsum(h1, st_ref[0:8, :], 1.0)


def _ka(hns, aggs, w1, b1, epsv):
    return pl.pallas_call(
        _ka_body,
        grid=(NB,),
        in_specs=[
            pl.BlockSpec((2, BN, HH), lambda i: (0, i, 0)),
            pl.BlockSpec((2, BN, HH), lambda i: (0, i, 0)),
            pl.BlockSpec((H, H2), lambda i: (0, 0)),
            pl.BlockSpec((1, H2), lambda i: (0, 0)),
            pl.BlockSpec(memory_space=pltpu.SMEM),
        ],
        out_specs=[
            pl.BlockSpec((BN, H2), lambda i: (i, 0)),
            pl.BlockSpec((10, H2), lambda i: (0, 0)),
        ],
        out_shape=[
            jax.ShapeDtypeStruct((N, H2), _F32),
            jax.ShapeDtypeStruct((10, H2), _F32),
        ],
    )(hns, aggs, w1, b1, epsv)


def _kb_body(h1_ref, st_ref, g_ref, be_ref, w_ref, b_ref, h2_ref, st2_ref):
    s = st_ref[...]
    mu = _foldmu(s[0:8]) / N
    dmu = mu - s[9:10, :]
    var = s[8:9, :] / N - dmu * dmu
    y = jnp.maximum((h1_ref[...] - mu) / jnp.sqrt(var + 1e-5) * g_ref[...]
                    + be_ref[...], 0.0)
    h2 = _dotd(y, w_ref[...]) + b_ref[...]
    h2_ref[...] = h2

    @pl.when(pl.program_id(0) == 0)
    def _():
        st2_ref[...] = jnp.zeros_like(st2_ref)
        st2_ref[9:10, :] = jnp.mean(h2, axis=0, keepdims=True)

    sh = st2_ref[9:10, :]
    d = h2 - sh
    st2_ref[8:9, :] += jnp.sum(d * d, axis=0, keepdims=True)
    st2_ref[0:8, :] = _groupsum(h2, st2_ref[0:8, :], 1.0)


def _kb(h1, st, g1, be1, w2, b2):
    return pl.pallas_call(
        _kb_body,
        grid=(NB,),
        in_specs=[
            pl.BlockSpec((BN, H2), lambda i: (i, 0)),
            pl.BlockSpec((10, H2), lambda i: (0, 0)),
            pl.BlockSpec((1, H2), lambda i: (0, 0)),
            pl.BlockSpec((1, H2), lambda i: (0, 0)),
            pl.BlockSpec((H2, H), lambda i: (0, 0)),
            pl.BlockSpec((1, H), lambda i: (0, 0)),
        ],
        out_specs=[
            pl.BlockSpec((BN, H), lambda i: (i, 0)),
            pl.BlockSpec((10, H), lambda i: (0, 0)),
        ],
        out_shape=[
            jax.ShapeDtypeStruct((N, H), _F32),
            jax.ShapeDtypeStruct((10, H), _F32),
        ],
    )(h1, st, g1, be1, w2, b2)


def _kc_body(h2_ref, st_ref, g_ref, be_ref, gid_ref, hn_ref, pool_ref, cnt_ref):
    s = st_ref[...]
    mu = _foldmu(s[0:8]) / N
    dmu = mu - s[9:10, :]
    var = s[8:9, :] / N - dmu * dmu
    y = jnp.maximum((h2_ref[...] - mu) / jnp.sqrt(var + 1e-5) * g_ref[...]
                    + be_ref[...], 0.0)
    hn_ref[0] = y[:, :HH]
    hn_ref[1] = y[:, HH:]
    oh = (gid_ref[...] == lax.broadcasted_iota(jnp.int32, (BN, B), 1)).astype(_F32)

    @pl.when(pl.program_id(0) == 0)
    def _():
        pool_ref[...] = jnp.zeros_like(pool_ref)
        cnt_ref[...] = jnp.zeros_like(cnt_ref)

    pool_ref[...] += lax.dot_general(oh, y, (((0,), (0,)), ((), ())),
                                     precision=_HI, preferred_element_type=_F32)
    cnt_ref[...] += jnp.sum(oh, axis=0, keepdims=True)


def _kc(h2, st2, g2, be2, gid2d):
    return pl.pallas_call(
        _kc_body,
        grid=(NB,),
        in_specs=[
            pl.BlockSpec((BN, H), lambda i: (i, 0)),
            pl.BlockSpec((10, H), lambda i: (0, 0)),
            pl.BlockSpec((1, H), lambda i: (0, 0)),
            pl.BlockSpec((1, H), lambda i: (0, 0)),
            pl.BlockSpec((BN, 1), lambda i: (i, 0)),
        ],
        out_specs=[
            pl.BlockSpec((2, BN, HH), lambda i: (0, i, 0)),
            pl.BlockSpec((B, H), lambda i: (0, 0)),
            pl.BlockSpec((1, B), lambda i: (0, 0)),
        ],
        out_shape=[
            jax.ShapeDtypeStruct((2, N, HH), _F32),
            jax.ShapeDtypeStruct((B, H), _F32),
            jax.ShapeDtypeStruct((1, B), _F32),
        ],
    )(h2, st2, g2, be2, gid2d)


def _kv_body(p_ref, ve_ref, w1_ref, b1_ref, g1_ref, be1_ref,
             w2_ref, b2_ref, g2_ref, be2_ref, out_ref):
    v = p_ref[...] + ve_ref[...]
    a = _dotd(v, w1_ref[...]) + b1_ref[...]
    mu = jnp.mean(a, axis=0, keepdims=True)
    d = a - mu
    var = jnp.mean(d * d, axis=0, keepdims=True)
    a = jnp.maximum(d / jnp.sqrt(var + 1e-5) * g1_ref[...] + be1_ref[...], 0.0)
    a = _dotd(a, w2_ref[...]) + b2_ref[...]
    mu = jnp.mean(a, axis=0, keepdims=True)
    d = a - mu
    var = jnp.mean(d * d, axis=0, keepdims=True)
    a = d / jnp.sqrt(var + 1e-5) * g2_ref[...] + be2_ref[...]
    out_ref[...] = jnp.maximum(a, 0.0)


def _kv(pooled, v_emb, w1, b1, g1, be1, w2, b2, g2, be2):
    specs = [
        pl.BlockSpec((B, H), lambda: (0, 0)),
        pl.BlockSpec((B, H), lambda: (0, 0)),
        pl.BlockSpec((H, H2), lambda: (0, 0)),
        pl.BlockSpec((1, H2), lambda: (0, 0)),
        pl.BlockSpec((1, H2), lambda: (0, 0)),
        pl.BlockSpec((1, H2), lambda: (0, 0)),
        pl.BlockSpec((H2, H), lambda: (0, 0)),
        pl.BlockSpec((1, H), lambda: (0, 0)),
        pl.BlockSpec((1, H), lambda: (0, 0)),
        pl.BlockSpec((1, H), lambda: (0, 0)),
    ]
    return pl.pallas_call(
        _kv_body,
        in_specs=specs,
        out_specs=pl.BlockSpec((B, H), lambda: (0, 0)),
        out_shape=jax.ShapeDtypeStruct((B, H), _F32),
    )(pooled, v_emb, w1, b1, g1, be1, w2, b2, g2, be2)


def _kf_body(p_ref, c_ref, w_ref, b_ref, out_ref):
    r = 1.0 / jnp.maximum(c_ref[...], 1.0)
    eye = (lax.broadcasted_iota(jnp.int32, (B, B), 0)
           == lax.broadcasted_iota(jnp.int32, (B, B), 1)).astype(_F32)
    pm = _dot(eye * r, p_ref[...])
    out_ref[...] = _dotd(pm, w_ref[...]) + b_ref[...]


def _kf(pooled, counts, wp, bp):
    return pl.pallas_call(
        _kf_body,
        in_specs=[
            pl.BlockSpec((B, H), lambda: (0, 0)),
            pl.BlockSpec((1, B), lambda: (0, 0)),
            pl.BlockSpec((H, NOUT), lambda: (0, 0)),
            pl.BlockSpec((1, NOUT), lambda: (0, 0)),
        ],
        out_specs=pl.BlockSpec((B, NOUT), lambda: (0, 0)),
        out_shape=jax.ShapeDtypeStruct((B, NOUT), _F32),
    )(pooled, counts, wp, bp)


def _edge_body(hn_hbm, t_hbm, src_hbm, dst_hbm, code_hbm, out_hbm,
               acc, srcv0, dstv0, codev0, srcv1, dstv1, codev1,
               hrow0, trow0, hrow1, trow1, zbuf,
               sh0, st0, sh1, st1):
    c = lax.axis_index("c")
    s = lax.axis_index("s")
    hnc = hn_hbm.at[c]
    tc_ = t_hbm.at[c]

    def _zb(i, carry):
        r = i // 8
        k = (i % 8) * 16
        zbuf[r, pl.ds(k, 16)] = jnp.zeros((16,), _F32)
        return carry

    lax.fori_loop(0, ZR * 8, _zb, 0)

    def _zc(i, carry):
        pltpu.sync_copy(zbuf, acc.at[pl.ds(s * RPS + i * ZR, ZR)])
        return carry

    lax.fori_loop(0, RPS // ZR, _zc, 0)
    plsc.subcore_barrier()

    bufs = ((hrow0, trow0, sh0, st0, srcv0, dstv0, codev0),
            (hrow1, trow1, sh1, st1, srcv1, dstv1, codev1))

    def _issue(j, b):
        hr, tr, sh, st, sv, dv, cv = bufs[b]
        base = s * EPT + j * CK
        pltpu.sync_copy(src_hbm.at[pl.ds(base, CK)], sv)
        pltpu.sync_copy(code_hbm.at[pl.ds(base, CK)], cv)
        pltpu.sync_copy(dst_hbm.at[pl.ds(base, CK)], dv)
        pltpu.async_copy(hnc.at[sv], hr, sh)
        pltpu.async_copy(tc_.at[cv], tr, st)

    def _drain(j, b):
        hr, tr, sh, st, sv, dv, cv = bufs[b]
        pltpu.make_async_copy(hnc.at[sv], hr, sh).wait()
        pltpu.make_async_copy(tc_.at[cv], tr, st).wait()

        def _ce(e, cr):
            for g in range(8):
                sl = pl.ds(g * 16, 16)
                hr[e, sl] = jnp.maximum(hr[e, sl] + tr[e, sl], 0.0)
            return cr

        lax.fori_loop(0, CK, _ce, 0)
        pltpu.sync_copy(hr, acc.at[dv], add=True)

    _issue(0, 0)

    def _pair(i, carry):
        _issue(2 * i + 1, 1)
        _drain(2 * i, 0)
        _issue(2 * i + 2, 0)
        _drain(2 * i + 1, 1)
        return carry

    lax.fori_loop(0, (NCH - 1) // 2, _pair, 0)
    _drain(NCH - 1, 0)

    plsc.subcore_barrier()
    pltpu.sync_copy(acc.at[pl.ds(s * RPS, RPS)],
                    out_hbm.at[pl.ds(c * NP + s * RPS, RPS)])


@functools.cache
def _edge_kernel():
    mesh = plsc.VectorSubcoreMesh(core_axis_name="c", subcore_axis_name="s")
    return pl.kernel(
        _edge_body,
        out_type=jax.ShapeDtypeStruct((2 * NP, HH), _F32),
        mesh=mesh,
        scratch_types=[
            pltpu.VMEM_SHARED((NP, HH), _F32),
            pltpu.VMEM((CK,), jnp.int32),
            pltpu.VMEM((CK,), jnp.int32),
            pltpu.VMEM((CK,), jnp.int32),
            pltpu.VMEM((CK,), jnp.int32),
            pltpu.VMEM((CK,), jnp.int32),
            pltpu.VMEM((CK,), jnp.int32),
            pltpu.VMEM((CK, HH), _F32),
            pltpu.VMEM((CK, HH), _F32),
            pltpu.VMEM((CK, HH), _F32),
            pltpu.VMEM((CK, HH), _F32),
            pltpu.VMEM((ZR, HH), _F32),
            pltpu.SemaphoreType.DMA,
            pltpu.SemaphoreType.DMA,
            pltpu.SemaphoreType.DMA,
            pltpu.SemaphoreType.DMA,
        ],
    )


def _edge_call(hn3, tab3, src3, dst3, code3):
    return _edge_kernel()(hn3, tab3, src3, dst3, code3)


def kernel(x, x_e, edge_index, node_graph_id, W_atom, W_bond, eps,
           gin_W1, gin_b1, gin_g1, gin_be1, gin_W2, gin_b2, gin_g2, gin_be2,
           v_W1, v_b1, v_g1, v_be1, v_W2, v_b2, v_g2, v_be2,
           v_emb_w, Wp, bp):
    x = x.astype(jnp.int32)
    x_e = x_e.astype(jnp.int32)
    src = edge_index[0].astype(jnp.int32)
    dst = edge_index[1].astype(jnp.int32)
    gid2d = node_graph_id.astype(jnp.int32).reshape(N, 1)

    w_atom_flat = W_atom.reshape(NAF * AV, H)
    cc = jnp.arange(NTAB)
    tab = (W_bond[:, 0, cc // 25] + W_bond[:, 1, (cc // 5) % 5]
           + W_bond[:, 2, cc % 5])
    tab_split = tab.reshape(L, NTAB, 2, HH).transpose(0, 2, 1, 3)

    codes = _edge_codes(x_e.T.reshape(NBF, 1250, 128)).reshape(E)

    hns = _atom_embed(x, w_atom_flat)
    v_emb = jnp.broadcast_to(v_emb_w, (B, H))

    pooled = None
    counts = None
    for i in range(L):
        hn2s = _pre(hns, gid2d, v_emb)
        aggf = _edge_call(hn2s, tab_split[i], src, dst, codes)
        aggs = aggf.reshape(2, NP, HH)
        h1, st1 = _ka(hn2s, aggs, gin_W1[i], gin_b1[i].reshape(1, H2),
                      eps[i].reshape(1, 1))
        h2, st2 = _kb(h1, st1, gin_g1[i].reshape(1, H2),
                      gin_be1[i].reshape(1, H2), gin_W2[i],
                      gin_b2[i].reshape(1, H))
        hns, pooled, counts = _kc(h2, st2, gin_g2[i].reshape(1, H),
                                  gin_be2[i].reshape(1, H), gid2d)
        if i < L - 1:
            v_emb = _kv(pooled, v_emb, v_W1[i], v_b1[i].reshape(1, H2),
                        v_g1[i].reshape(1, H2), v_be1[i].reshape(1, H2),
                        v_W2[i], v_b2[i].reshape(1, H),
                        v_g2[i].reshape(1, H), v_be2[i].reshape(1, H))

    return _kf(pooled, counts, Wp, bp.reshape(1, NOUT))
